# baseline (speedup 1.0000x reference)
import functools

import jax
import jax.numpy as jnp
from jax import lax
from jax.experimental import pallas as pl
from jax.experimental.pallas import tpu as pltpu
from jax.experimental.pallas import tpu_sc as plsc

EMBED_DIM = 11
NUM_ROWS = 100000
BATCH = 16384

NC = 2
NS = 16
NW = NC * NS
B_PER_W = BATCH // NW


def _gather_body(idx_hbm, tflat_hbm, outT_hbm, idx_v, lists_v, rows_v, sem):
    wid = lax.axis_index("s") * NC + lax.axis_index("c")
    base = wid * B_PER_W
    pltpu.sync_copy(idx_hbm.at[pl.ds(base, B_PER_W)], idx_v)
    for w in range(B_PER_W // 16):
        iw = idx_v[pl.ds(w * 16, 16)]
        for c in range(EMBED_DIM):
            lists_v[pl.ds(c * B_PER_W + w * 16, 16)] = iw + c * NUM_ROWS
    pltpu.async_copy(tflat_hbm.at[lists_v], rows_v, sem).wait()
    for c in range(EMBED_DIM):
        pltpu.sync_copy(
            rows_v.at[pl.ds(c * B_PER_W, B_PER_W)],
            outT_hbm.at[c, pl.ds(base, B_PER_W)],
        )


@jax.jit
def _gather(idx, tflat):
    mesh = plsc.VectorSubcoreMesh(core_axis_name="c", subcore_axis_name="s")
    run = functools.partial(
        pl.kernel,
        mesh=mesh,
        out_type=jax.ShapeDtypeStruct((EMBED_DIM, BATCH), jnp.float32),
        scratch_types=[
            pltpu.VMEM((B_PER_W,), jnp.int32),
            pltpu.VMEM((EMBED_DIM * B_PER_W,), jnp.int32),
            pltpu.VMEM((EMBED_DIM * B_PER_W,), jnp.float32),
            pltpu.SemaphoreType.DMA,
        ],
        compiler_params=pltpu.CompilerParams(use_tc_tiling_on_sc=False),
    )(_gather_body)
    return run(idx, tflat).T


def kernel(device_num_tensor, table):
    idx = device_num_tensor.astype(jnp.int32)
    tflat = table.T.reshape(-1)
    return _gather(idx, tflat)


# speedup vs baseline: 1.0041x; 1.0041x over previous
import functools

import jax
import jax.numpy as jnp
from jax import lax
from jax.experimental import pallas as pl
from jax.experimental.pallas import tpu as pltpu
from jax.experimental.pallas import tpu_sc as plsc

EMBED_DIM = 11
NUM_ROWS = 100000
BATCH = 16384

NC = 2
NS = 16
NW = NC * NS
B_PER_W = BATCH // NW


def _gather_body(idx_hbm, tflat_hbm, outT_hbm, idx_v, lists_v, rows_v, sem):
    wid = lax.axis_index("s") * NC + lax.axis_index("c")
    base = wid * B_PER_W
    pltpu.sync_copy(idx_hbm.at[pl.ds(base, B_PER_W)], idx_v)
    copies = []
    for c in range(EMBED_DIM):
        for w in range(B_PER_W // 16):
            iw = idx_v[pl.ds(w * 16, 16)]
            lists_v[pl.ds(c * B_PER_W + w * 16, 16)] = iw + c * NUM_ROWS
        copies.append(
            pltpu.async_copy(
                tflat_hbm.at[lists_v.at[pl.ds(c * B_PER_W, B_PER_W)]],
                rows_v.at[pl.ds(c * B_PER_W, B_PER_W)],
                sem,
            )
        )
    for c, cp in enumerate(copies):
        cp.wait()
        pltpu.sync_copy(
            rows_v.at[pl.ds(c * B_PER_W, B_PER_W)],
            outT_hbm.at[c, pl.ds(base, B_PER_W)],
        )


@jax.jit
def _gather(idx, tflat):
    mesh = plsc.VectorSubcoreMesh(core_axis_name="c", subcore_axis_name="s")
    run = functools.partial(
        pl.kernel,
        mesh=mesh,
        out_type=jax.ShapeDtypeStruct((EMBED_DIM, BATCH), jnp.float32),
        scratch_types=[
            pltpu.VMEM((B_PER_W,), jnp.int32),
            pltpu.VMEM((EMBED_DIM * B_PER_W,), jnp.int32),
            pltpu.VMEM((EMBED_DIM * B_PER_W,), jnp.float32),
            pltpu.SemaphoreType.DMA,
        ],
        compiler_params=pltpu.CompilerParams(use_tc_tiling_on_sc=False),
    )(_gather_body)
    return run(idx, tflat).T


def kernel(device_num_tensor, table):
    idx = device_num_tensor.astype(jnp.int32)
    tflat = table.T.reshape(-1)
    return _gather(idx, tflat)
